# baseline (speedup 1.0000x reference)
import functools
import math

import jax
import jax.numpy as jnp
from jax import lax
from jax.experimental import pallas as pl
from jax.experimental.pallas import tpu as pltpu

S = 4096
E = 8
CAPACITY = math.ceil(2 * 1.25 * S / E)
CS = 256
TB = 256
GRID = S // TB


def _meta_kernel(x_ref, e1_ref, e2_ref, p1_ref, p2_ref, w1_ref, w2_ref,
                 cnt_ref):
    x = x_ref[...]
    xmax = jnp.max(x, axis=1, keepdims=True)
    ex = jnp.exp(x - xmax)
    logits = ex / jnp.sum(ex, axis=1, keepdims=True)

    e_iota = lax.broadcasted_iota(jnp.int32, (S, E), 1)
    m1v = jnp.max(logits, axis=1, keepdims=True)
    i1 = jnp.min(jnp.where(logits == m1v, e_iota, E), axis=1, keepdims=True)
    masked = jnp.where(e_iota == i1, -jnp.inf, logits)
    m2v = jnp.max(masked, axis=1, keepdims=True)
    i2 = jnp.min(jnp.where(masked == m2v, e_iota, E), axis=1, keepdims=True)

    m1 = (e_iota == i1).astype(jnp.float32)
    m2 = (e_iota == i2).astype(jnp.float32)

    tri = (lax.broadcasted_iota(jnp.int32, (CS, CS), 0)
           >= lax.broadcasted_iota(jnp.int32, (CS, CS), 1)).astype(jnp.float32)
    run1 = jnp.zeros((1, E), jnp.float32)
    run2 = jnp.zeros((1, E), jnp.float32)
    c1_blocks = []
    c2_blocks = []
    for j in range(S // CS):
        blk1 = m1[j * CS:(j + 1) * CS, :]
        blk2 = m2[j * CS:(j + 1) * CS, :]
        c1 = lax.dot(tri, blk1, preferred_element_type=jnp.float32) + run1
        c2 = lax.dot(tri, blk2, preferred_element_type=jnp.float32) + run2
        run1 = c1[CS - 1:CS, :]
        run2 = c2[CS - 1:CS, :]
        c1_blocks.append(c1)
        c2_blocks.append(c2)
    cum1 = jnp.concatenate(c1_blocks, axis=0)
    cum2 = jnp.concatenate(c2_blocks, axis=0)
    total1 = run1
    total2 = run2

    loc1 = cum1 - 1.0
    loc2 = cum2 - 1.0 + total1

    cap = jnp.float32(CAPACITY)
    k1 = m1 * (loc1 < cap).astype(jnp.float32)
    k2 = m2 * (loc2 < cap).astype(jnp.float32)

    w1 = jnp.sum(k1 * logits, axis=1, keepdims=True)
    w2 = jnp.sum(k2 * logits, axis=1, keepdims=True)
    p1 = jnp.sum(k1 * loc1, axis=1, keepdims=True)
    p2 = jnp.sum(k2 * loc2, axis=1, keepdims=True)

    e1_ref[...] = i1
    e2_ref[...] = i2
    p1_ref[...] = p1.astype(jnp.int32)
    p2_ref[...] = p2.astype(jnp.int32)
    w1_ref[...] = w1
    w2_ref[...] = w2
    cnt_ref[...] = (total1 + total2).astype(jnp.int32)


def _expand_kernel(e1_ref, e2_ref, p1_ref, p2_ref, w1_ref, w2_ref,
                   cw_ref, sm_ref):
    e1 = e1_ref[...].reshape(TB, 1, 1)
    e2 = e2_ref[...].reshape(TB, 1, 1)
    p1 = p1_ref[...].reshape(TB, 1, 1)
    p2 = p2_ref[...].reshape(TB, 1, 1)
    w1 = w1_ref[...].reshape(TB, 1, 1)
    w2 = w2_ref[...].reshape(TB, 1, 1)
    e_io = lax.broadcasted_iota(jnp.int32, (TB, E, CAPACITY), 1)
    c_io = lax.broadcasted_iota(jnp.int32, (TB, E, CAPACITY), 2)
    hit1 = (e_io == e1) & (c_io == p1)
    hit2 = (e_io == e2) & (c_io == p2)
    cw = jnp.where(hit1, w1, 0.0) + jnp.where(hit2, w2, 0.0)
    cw_ref[...] = cw
    sm_ref[...] = cw != 0.0


@jax.jit
def kernel(inputs):
    meta_shapes = (
        jax.ShapeDtypeStruct((S, 1), jnp.int32),
        jax.ShapeDtypeStruct((S, 1), jnp.int32),
        jax.ShapeDtypeStruct((S, 1), jnp.int32),
        jax.ShapeDtypeStruct((S, 1), jnp.int32),
        jax.ShapeDtypeStruct((S, 1), jnp.float32),
        jax.ShapeDtypeStruct((S, 1), jnp.float32),
        jax.ShapeDtypeStruct((1, E), jnp.int32),
    )
    e1, e2, p1, p2, w1, w2, cnt = pl.pallas_call(
        _meta_kernel,
        out_shape=meta_shapes,
    )(inputs)

    tok_spec = pl.BlockSpec((TB, 1), lambda i: (i, 0))
    cw, sm = pl.pallas_call(
        _expand_kernel,
        grid=(GRID,),
        in_specs=[tok_spec] * 6,
        out_specs=(
            pl.BlockSpec((TB, E, CAPACITY), lambda i: (i, 0, 0)),
            pl.BlockSpec((TB, E, CAPACITY), lambda i: (i, 0, 0)),
        ),
        out_shape=(
            jax.ShapeDtypeStruct((S, E, CAPACITY), jnp.float32),
            jax.ShapeDtypeStruct((S, E, CAPACITY), jnp.bool_),
        ),
    )(e1, e2, p1, p2, w1, w2)

    return (cw, sm, cnt.reshape(E))


# speedup vs baseline: 1.0284x; 1.0284x over previous
import functools
import math

import jax
import jax.numpy as jnp
from jax import lax
from jax.experimental import pallas as pl
from jax.experimental.pallas import tpu as pltpu

S = 4096
E = 8
CAPACITY = math.ceil(2 * 1.25 * S / E)
CS = 256
TB = 128
GRID = S // TB


def _meta_kernel(x_ref, e1_ref, e2_ref, p1_ref, p2_ref, w1_ref, w2_ref,
                 cnt_ref):
    x = x_ref[...]
    xmax = jnp.max(x, axis=1, keepdims=True)
    ex = jnp.exp(x - xmax)
    logits = ex / jnp.sum(ex, axis=1, keepdims=True)

    e_iota = lax.broadcasted_iota(jnp.int32, (S, E), 1)
    m1v = jnp.max(logits, axis=1, keepdims=True)
    i1 = jnp.min(jnp.where(logits == m1v, e_iota, E), axis=1, keepdims=True)
    masked = jnp.where(e_iota == i1, -jnp.inf, logits)
    m2v = jnp.max(masked, axis=1, keepdims=True)
    i2 = jnp.min(jnp.where(masked == m2v, e_iota, E), axis=1, keepdims=True)

    m1 = (e_iota == i1).astype(jnp.float32)
    m2 = (e_iota == i2).astype(jnp.float32)

    tri = (lax.broadcasted_iota(jnp.int32, (CS, CS), 0)
           >= lax.broadcasted_iota(jnp.int32, (CS, CS), 1)).astype(jnp.float32)
    run1 = jnp.zeros((1, E), jnp.float32)
    run2 = jnp.zeros((1, E), jnp.float32)
    c1_blocks = []
    c2_blocks = []
    for j in range(S // CS):
        blk1 = m1[j * CS:(j + 1) * CS, :]
        blk2 = m2[j * CS:(j + 1) * CS, :]
        c1 = lax.dot(tri, blk1, preferred_element_type=jnp.float32) + run1
        c2 = lax.dot(tri, blk2, preferred_element_type=jnp.float32) + run2
        run1 = c1[CS - 1:CS, :]
        run2 = c2[CS - 1:CS, :]
        c1_blocks.append(c1)
        c2_blocks.append(c2)
    cum1 = jnp.concatenate(c1_blocks, axis=0)
    cum2 = jnp.concatenate(c2_blocks, axis=0)
    total1 = run1
    total2 = run2

    loc1 = cum1 - 1.0
    loc2 = cum2 - 1.0 + total1

    cap = jnp.float32(CAPACITY)
    k1 = m1 * (loc1 < cap).astype(jnp.float32)
    k2 = m2 * (loc2 < cap).astype(jnp.float32)

    w1 = jnp.sum(k1 * logits, axis=1, keepdims=True)
    w2 = jnp.sum(k2 * logits, axis=1, keepdims=True)
    p1 = jnp.sum(k1 * loc1, axis=1, keepdims=True)
    p2 = jnp.sum(k2 * loc2, axis=1, keepdims=True)

    e1_ref[...] = i1
    e2_ref[...] = i2
    p1_ref[...] = p1.astype(jnp.int32)
    p2_ref[...] = p2.astype(jnp.int32)
    w1_ref[...] = w1
    w2_ref[...] = w2
    cnt_ref[...] = (total1 + total2).astype(jnp.int32)


def _expand_kernel(e1_ref, e2_ref, p1_ref, p2_ref, w1_ref, w2_ref,
                   cw_ref, sm_ref):
    e1 = e1_ref[...].reshape(TB, 1, 1)
    e2 = e2_ref[...].reshape(TB, 1, 1)
    p1 = p1_ref[...].reshape(TB, 1, 1)
    p2 = p2_ref[...].reshape(TB, 1, 1)
    w1 = w1_ref[...].reshape(TB, 1, 1)
    w2 = w2_ref[...].reshape(TB, 1, 1)
    e_io = lax.broadcasted_iota(jnp.int32, (TB, E, 1), 1)
    we1 = jnp.where(e_io == e1, w1, 0.0)
    we2 = jnp.where(e_io == e2, w2, 0.0)
    c_io = lax.broadcasted_iota(jnp.int32, (TB, E, CAPACITY), 2)
    cw = jnp.where(c_io == p1, we1, 0.0) + jnp.where(c_io == p2, we2, 0.0)
    cw_ref[...] = cw
    sm_ref[...] = cw != 0.0


@jax.jit
def kernel(inputs):
    meta_shapes = (
        jax.ShapeDtypeStruct((S, 1), jnp.int32),
        jax.ShapeDtypeStruct((S, 1), jnp.int32),
        jax.ShapeDtypeStruct((S, 1), jnp.int32),
        jax.ShapeDtypeStruct((S, 1), jnp.int32),
        jax.ShapeDtypeStruct((S, 1), jnp.float32),
        jax.ShapeDtypeStruct((S, 1), jnp.float32),
        jax.ShapeDtypeStruct((1, E), jnp.int32),
    )
    e1, e2, p1, p2, w1, w2, cnt = pl.pallas_call(
        _meta_kernel,
        out_shape=meta_shapes,
    )(inputs)

    tok_spec = pl.BlockSpec((TB, 1), lambda i: (i, 0))
    cw, sm = pl.pallas_call(
        _expand_kernel,
        grid=(GRID,),
        in_specs=[tok_spec] * 6,
        out_specs=(
            pl.BlockSpec((TB, E, CAPACITY), lambda i: (i, 0, 0)),
            pl.BlockSpec((TB, E, CAPACITY), lambda i: (i, 0, 0)),
        ),
        out_shape=(
            jax.ShapeDtypeStruct((S, E, CAPACITY), jnp.float32),
            jax.ShapeDtypeStruct((S, E, CAPACITY), jnp.bool_),
        ),
    )(e1, e2, p1, p2, w1, w2)

    return (cw, sm, cnt.reshape(E))


# speedup vs baseline: 1.0594x; 1.0301x over previous
import math

import jax
import jax.numpy as jnp
from jax import lax
from jax.experimental import pallas as pl
from jax.experimental.pallas import tpu as pltpu

S = 4096
E = 8
CAPACITY = math.ceil(2 * 1.25 * S / E)
CS = 256
TB = 128
GRID = S // TB
NBUF = 2


def _meta_phase(x_ref, cnt_ref, e1_s, e2_s, p1_s, p2_s, w1_s, w2_s):
    x = x_ref[...]
    xmax = jnp.max(x, axis=1, keepdims=True)
    ex = jnp.exp(x - xmax)
    logits = ex / jnp.sum(ex, axis=1, keepdims=True)

    e_iota = lax.broadcasted_iota(jnp.int32, (S, E), 1)
    m1v = jnp.max(logits, axis=1, keepdims=True)
    i1 = jnp.min(jnp.where(logits == m1v, e_iota, E), axis=1, keepdims=True)
    masked = jnp.where(e_iota == i1, -jnp.inf, logits)
    m2v = jnp.max(masked, axis=1, keepdims=True)
    i2 = jnp.min(jnp.where(masked == m2v, e_iota, E), axis=1, keepdims=True)

    m1 = (e_iota == i1).astype(jnp.float32)
    m2 = (e_iota == i2).astype(jnp.float32)

    tri = (lax.broadcasted_iota(jnp.int32, (CS, CS), 0)
           >= lax.broadcasted_iota(jnp.int32, (CS, CS), 1)).astype(jnp.float32)
    run1 = jnp.zeros((1, E), jnp.float32)
    run2 = jnp.zeros((1, E), jnp.float32)
    c1_blocks = []
    c2_blocks = []
    for j in range(S // CS):
        blk1 = m1[j * CS:(j + 1) * CS, :]
        blk2 = m2[j * CS:(j + 1) * CS, :]
        c1 = lax.dot(tri, blk1, preferred_element_type=jnp.float32) + run1
        c2 = lax.dot(tri, blk2, preferred_element_type=jnp.float32) + run2
        run1 = c1[CS - 1:CS, :]
        run2 = c2[CS - 1:CS, :]
        c1_blocks.append(c1)
        c2_blocks.append(c2)
    cum1 = jnp.concatenate(c1_blocks, axis=0)
    cum2 = jnp.concatenate(c2_blocks, axis=0)
    total1 = run1
    total2 = run2

    loc1 = cum1 - 1.0
    loc2 = cum2 - 1.0 + total1

    cap = jnp.float32(CAPACITY)
    k1 = m1 * (loc1 < cap).astype(jnp.float32)
    k2 = m2 * (loc2 < cap).astype(jnp.float32)

    e1_s[...] = i1
    e2_s[...] = i2
    p1_s[...] = jnp.sum(k1 * loc1, axis=1, keepdims=True).astype(jnp.int32)
    p2_s[...] = jnp.sum(k2 * loc2, axis=1, keepdims=True).astype(jnp.int32)
    w1_s[...] = jnp.sum(k1 * logits, axis=1, keepdims=True)
    w2_s[...] = jnp.sum(k2 * logits, axis=1, keepdims=True)
    cnt_ref[...] = (total1 + total2).astype(jnp.int32)


def _fused_kernel(x_ref, cw_hbm, sm_ref, cnt_ref,
                  e1_s, e2_s, p1_s, p2_s, w1_s, w2_s,
                  cw_bufs, sems):
    j = pl.program_id(0)

    @pl.when(j == 0)
    def _():
        _meta_phase(x_ref, cnt_ref, e1_s, e2_s, p1_s, p2_s, w1_s, w2_s)

    slot = lax.rem(j, NBUF)

    def cw_copy(blk, s):
        return pltpu.make_async_copy(
            cw_bufs.at[s], cw_hbm.at[pl.ds(blk * TB, TB)], sems.at[s])

    @pl.when(j >= NBUF)
    def _():
        cw_copy(j - NBUF, slot).wait()

    tok = pl.ds(j * TB, TB)
    e1 = e1_s[tok, :].reshape(TB, 1, 1)
    e2 = e2_s[tok, :].reshape(TB, 1, 1)
    p1 = p1_s[tok, :].reshape(TB, 1, 1)
    p2 = p2_s[tok, :].reshape(TB, 1, 1)
    w1 = w1_s[tok, :].reshape(TB, 1, 1)
    w2 = w2_s[tok, :].reshape(TB, 1, 1)
    e_io = lax.broadcasted_iota(jnp.int32, (TB, E, 1), 1)
    we1 = jnp.where(e_io == e1, w1, 0.0)
    we2 = jnp.where(e_io == e2, w2, 0.0)
    c_io = lax.broadcasted_iota(jnp.int32, (TB, E, CAPACITY), 2)
    cw = jnp.where(c_io == p1, we1, 0.0) + jnp.where(c_io == p2, we2, 0.0)
    cw_bufs[slot] = cw
    sm_ref[...] = cw != 0.0

    cw_copy(j, slot).start()

    @pl.when(j == GRID - 1)
    def _():
        cw_copy(j - 1, lax.rem(j - 1, NBUF)).wait()
        cw_copy(j, slot).wait()


@jax.jit
def kernel(inputs):
    cw, sm, cnt = pl.pallas_call(
        _fused_kernel,
        grid=(GRID,),
        in_specs=[pl.BlockSpec((S, E), lambda i: (0, 0))],
        out_shape=(
            jax.ShapeDtypeStruct((S, E, CAPACITY), jnp.float32),
            jax.ShapeDtypeStruct((S, E, CAPACITY), jnp.bool_),
            jax.ShapeDtypeStruct((1, E), jnp.int32),
        ),
        out_specs=(
            pl.BlockSpec(memory_space=pltpu.MemorySpace.HBM),
            pl.BlockSpec((TB, E, CAPACITY), lambda i: (i, 0, 0)),
            pl.BlockSpec((1, E), lambda i: (0, 0)),
        ),
        scratch_shapes=(
            [pltpu.VMEM((S, 1), jnp.int32)] * 4
            + [pltpu.VMEM((S, 1), jnp.float32)] * 2
            + [pltpu.VMEM((NBUF, TB, E, CAPACITY), jnp.float32),
               pltpu.SemaphoreType.DMA((NBUF,))]
        ),
    )(inputs)
    return (cw, sm, cnt.reshape(E))
